# baseline (speedup 1.0000x reference)
import functools

import jax
import jax.numpy as jnp
from jax import lax
from jax.experimental import pallas as pl
from jax.experimental.pallas import tpu as pltpu
from jax.experimental.pallas import tpu_sc as plsc

NTOK = 4 * 8192
D = 1024
NC, NS = 2, 16
NW = NC * NS
PER_W = NTOK // NW
ROWS = 1024
NB = NTOK // ROWS


@functools.lru_cache(maxsize=1)
def _make_flag_gather():
    mesh = plsc.VectorSubcoreMesh(core_axis_name="c", subcore_axis_name="s")

    @functools.partial(
        pl.kernel,
        mesh=mesh,
        out_type=jax.ShapeDtypeStruct((NTOK,), jnp.int32),
        scratch_types=[
            pltpu.VMEM((PER_W,), jnp.int32),
            pltpu.VMEM((PER_W,), jnp.int32),
            pltpu.SemaphoreType.DMA,
        ],
    )
    def gather_flags(ids_hbm, table_hbm, out_hbm, idx_v, flags_v, sem):
        wid = lax.axis_index("s") * NC + lax.axis_index("c")
        base = wid * PER_W
        pltpu.sync_copy(ids_hbm.at[pl.ds(base, PER_W)], idx_v)
        pltpu.async_copy(table_hbm.at[idx_v], flags_v, sem).wait()
        pltpu.sync_copy(flags_v, out_hbm.at[pl.ds(base, PER_W)])

    return gather_flags


def _tc_body(f_ref, w_ref, e_ref, o_ref):
    f = f_ref[...].astype(jnp.float32)
    w0 = w_ref[0:1, :]
    w1 = w_ref[1:2, :]
    o_ref[...] = e_ref[...] + (w0 + f * (w1 - w0))


def kernel(subword_embeds, token_ids, is_continuation, cont_emb_weight):
    vocab = is_continuation.shape[0] - 1
    ids = jnp.minimum(token_ids, vocab).astype(jnp.int32).reshape(NTOK)
    table = is_continuation.astype(jnp.int32)

    flags = _make_flag_gather()(ids, table)

    e2d = subword_embeds.reshape(NTOK, D)
    out = pl.pallas_call(
        _tc_body,
        grid=(NB,),
        in_specs=[
            pl.BlockSpec((ROWS, 1), lambda i: (i, 0)),
            pl.BlockSpec((2, D), lambda i: (0, 0)),
            pl.BlockSpec((ROWS, D), lambda i: (i, 0)),
        ],
        out_specs=pl.BlockSpec((ROWS, D), lambda i: (i, 0)),
        out_shape=jax.ShapeDtypeStruct((NTOK, D), jnp.float32),
        compiler_params=pltpu.CompilerParams(
            dimension_semantics=("arbitrary",),
        ),
    )(flags.reshape(NTOK, 1), cont_emb_weight.astype(jnp.float32), e2d)
    return out.reshape(subword_embeds.shape)


# speedup vs baseline: 1.0141x; 1.0141x over previous
import functools

import jax
import jax.numpy as jnp
from jax import lax
from jax.experimental import pallas as pl
from jax.experimental.pallas import tpu as pltpu
from jax.experimental.pallas import tpu_sc as plsc

NTOK = 4 * 8192
D = 1024
NC, NS = 2, 16
NW = NC * NS
PER_W = NTOK // NW
CH = 1024
NCH = NTOK // CH
K = 4


@functools.lru_cache(maxsize=1)
def _make_flag_gather():
    mesh = plsc.VectorSubcoreMesh(core_axis_name="c", subcore_axis_name="s")

    @functools.partial(
        pl.kernel,
        mesh=mesh,
        out_type=jax.ShapeDtypeStruct((NTOK,), jnp.int32),
        scratch_types=[
            pltpu.VMEM((PER_W,), jnp.int32),
            pltpu.VMEM((PER_W,), jnp.int32),
            pltpu.SemaphoreType.DMA,
        ],
    )
    def gather_flags(ids_hbm, table_hbm, out_hbm, idx_v, flags_v, sem):
        wid = lax.axis_index("s") * NC + lax.axis_index("c")
        base = wid * PER_W
        pltpu.sync_copy(ids_hbm.at[pl.ds(base, PER_W)], idx_v)
        pltpu.async_copy(table_hbm.at[idx_v], flags_v, sem).wait()
        pltpu.sync_copy(flags_v, out_hbm.at[pl.ds(base, PER_W)])

    return gather_flags


def _tc_body(f_hbm, w_ref, e_hbm, o_hbm, ebufs, obufs, fbufs,
             esems, fsems, osems):
    def start_read(g, slot):
        pltpu.make_async_copy(
            e_hbm.at[pl.ds(g * CH, CH), :], ebufs.at[slot], esems.at[slot]
        ).start()
        pltpu.make_async_copy(
            f_hbm.at[pl.ds(g * CH, CH), :], fbufs.at[slot], fsems.at[slot]
        ).start()

    def out_copy(g, slot):
        return pltpu.make_async_copy(
            obufs.at[slot], o_hbm.at[pl.ds(g * CH, CH), :], osems.at[slot]
        )

    for slot in range(K):
        start_read(slot, slot)

    w0 = w_ref[0:1, :]
    dw = w_ref[1:2, :] - w0

    for g in range(NCH):
        slot = g % K
        pltpu.make_async_copy(
            e_hbm.at[pl.ds(g * CH, CH), :], ebufs.at[slot], esems.at[slot]
        ).wait()
        pltpu.make_async_copy(
            f_hbm.at[pl.ds(g * CH, CH), :], fbufs.at[slot], fsems.at[slot]
        ).wait()
        if g >= K:
            out_copy(g - K, slot).wait()
        f = fbufs[slot].astype(jnp.float32)
        obufs[slot] = ebufs[slot] + (w0 + f * dw)
        out_copy(g, slot).start()
        nxt = g + K
        if nxt < NCH:
            start_read(nxt, slot)

    for g in range(NCH - K, NCH):
        out_copy(g, g % K).wait()


def kernel(subword_embeds, token_ids, is_continuation, cont_emb_weight):
    vocab = is_continuation.shape[0] - 1
    ids = jnp.minimum(token_ids, vocab).astype(jnp.int32).reshape(NTOK)
    table = is_continuation.astype(jnp.int32)

    flags = _make_flag_gather()(ids, table)

    e2d = subword_embeds.reshape(NTOK, D)
    out = pl.pallas_call(
        _tc_body,
        in_specs=[
            pl.BlockSpec(memory_space=pl.ANY),
            pl.BlockSpec((2, D), lambda: (0, 0)),
            pl.BlockSpec(memory_space=pl.ANY),
        ],
        out_specs=pl.BlockSpec(memory_space=pl.ANY),
        out_shape=jax.ShapeDtypeStruct((NTOK, D), jnp.float32),
        scratch_shapes=[
            pltpu.VMEM((K, CH, D), jnp.float32),
            pltpu.VMEM((K, CH, D), jnp.float32),
            pltpu.VMEM((K, CH, 1), jnp.int32),
            pltpu.SemaphoreType.DMA((K,)),
            pltpu.SemaphoreType.DMA((K,)),
            pltpu.SemaphoreType.DMA((K,)),
        ],
    )(flags.reshape(NTOK, 1), cont_emb_weight.astype(jnp.float32), e2d)
    return out.reshape(subword_embeds.shape)


# speedup vs baseline: 1.1592x; 1.1431x over previous
import functools

import jax
import jax.numpy as jnp
from jax import lax
from jax.experimental import pallas as pl
from jax.experimental.pallas import tpu as pltpu
from jax.experimental.pallas import tpu_sc as plsc

NTOK = 4 * 8192
D = 1024
NC, NS = 2, 16
NW = NC * NS
PER_W = NTOK // NW
CH = 1024
NCH = NTOK // CH
K = 4


@functools.lru_cache(maxsize=1)
def _make_flag_gather():
    mesh = plsc.VectorSubcoreMesh(core_axis_name="c", subcore_axis_name="s")

    @functools.partial(
        pl.kernel,
        mesh=mesh,
        out_type=jax.ShapeDtypeStruct((NTOK,), jnp.int32),
        scratch_types=[
            pltpu.VMEM((PER_W,), jnp.int32),
            pltpu.VMEM((PER_W,), jnp.int32),
            pltpu.SemaphoreType.DMA,
        ],
    )
    def gather_flags(ids_hbm, table_hbm, out_hbm, idx_v, flags_v, sem):
        wid = lax.axis_index("s") * NC + lax.axis_index("c")
        base = wid * PER_W
        pltpu.sync_copy(ids_hbm.at[pl.ds(base, PER_W)], idx_v)
        pltpu.async_copy(table_hbm.at[idx_v], flags_v, sem).wait()
        pltpu.sync_copy(flags_v, out_hbm.at[pl.ds(base, PER_W)])

    return gather_flags


def _tc_body(f_hbm, w_ref, e_hbm, o_hbm, ebufs, obufs, fbufs,
             esems, fsems, osems):
    def start_read(g, slot):
        pltpu.make_async_copy(
            e_hbm.at[pl.ds(g * CH, CH), :], ebufs.at[slot], esems.at[slot]
        ).start()
        pltpu.make_async_copy(
            f_hbm.at[pl.ds(g * CH, CH)], fbufs.at[slot], fsems.at[slot]
        ).start()

    def out_copy(g, slot):
        return pltpu.make_async_copy(
            obufs.at[slot], o_hbm.at[pl.ds(g * CH, CH), :], osems.at[slot]
        )

    for slot in range(K):
        start_read(slot, slot)

    w0 = w_ref[0:1, :]
    dw = w_ref[1:2, :] - w0

    for g in range(NCH):
        slot = g % K
        pltpu.make_async_copy(
            e_hbm.at[pl.ds(g * CH, CH), :], ebufs.at[slot], esems.at[slot]
        ).wait()
        pltpu.make_async_copy(
            f_hbm.at[pl.ds(g * CH, CH)], fbufs.at[slot], fsems.at[slot]
        ).wait()
        if g >= K:
            out_copy(g - K, slot).wait()
        f = fbufs[slot].astype(jnp.float32).reshape(CH, 1)
        obufs[slot] = ebufs[slot] + (w0 + f * dw)
        out_copy(g, slot).start()
        nxt = g + K
        if nxt < NCH:
            start_read(nxt, slot)

    for g in range(NCH - K, NCH):
        out_copy(g, g % K).wait()


def kernel(subword_embeds, token_ids, is_continuation, cont_emb_weight):
    vocab = is_continuation.shape[0] - 1
    ids = jnp.minimum(token_ids, vocab).astype(jnp.int32).reshape(NTOK)
    table = is_continuation.astype(jnp.int32)

    flags = _make_flag_gather()(ids, table)

    e2d = subword_embeds.reshape(NTOK, D)
    out = pl.pallas_call(
        _tc_body,
        in_specs=[
            pl.BlockSpec(memory_space=pl.ANY),
            pl.BlockSpec((2, D), lambda: (0, 0)),
            pl.BlockSpec(memory_space=pl.ANY),
        ],
        out_specs=pl.BlockSpec(memory_space=pl.ANY),
        out_shape=jax.ShapeDtypeStruct((NTOK, D), jnp.float32),
        scratch_shapes=[
            pltpu.VMEM((K, CH, D), jnp.float32),
            pltpu.VMEM((K, CH, D), jnp.float32),
            pltpu.VMEM((K, CH), jnp.int32),
            pltpu.SemaphoreType.DMA((K,)),
            pltpu.SemaphoreType.DMA((K,)),
            pltpu.SemaphoreType.DMA((K,)),
        ],
    )(flags, cont_emb_weight.astype(jnp.float32), e2d)
    return out.reshape(subword_embeds.shape)
